# Initial kernel scaffold; baseline (speedup 1.0000x reference)
import jax
import jax.numpy as jnp
from jax.experimental import pallas as pl


def kernel(data, indices):
    raise NotImplementedError("write your pallas kernel here")


# speedup vs baseline: 1.0986x; 1.0986x over previous
import functools

import jax
import jax.numpy as jnp
from jax import lax
from jax.experimental import pallas as pl
from jax.experimental.pallas import tpu as pltpu
from jax.experimental.pallas import tpu_sc as plsc

_NC = 2
_NS = 16
_NW = _NC * _NS
_CS = 1280


def kernel(data, indices):
    batch, hist = indices.shape
    _, dim = data.shape
    n = batch * hist
    flat_idx = indices.reshape(n)
    b_per_w = n // _NW
    n_chunks = b_per_w // _CS

    mesh = plsc.VectorSubcoreMesh(core_axis_name="c", subcore_axis_name="s")

    @functools.partial(
        pl.kernel,
        mesh=mesh,
        compiler_params=pltpu.CompilerParams(use_tc_tiling_on_sc=False),
        out_type=jax.ShapeDtypeStruct((n, dim), jnp.float32),
        scratch_types=[
            pltpu.VMEM((_CS,), jnp.int32),
            pltpu.VMEM((_CS, dim), jnp.float32),
            pltpu.SemaphoreType.DMA,
        ],
    )
    def gather_k(table_hbm, idx_hbm, out_hbm, idx_v, rows_v, sem):
        wid = lax.axis_index("s") * _NC + lax.axis_index("c")
        base = wid * b_per_w

        @pl.loop(0, n_chunks)
        def _chunk(j):
            off = base + j * _CS
            pltpu.sync_copy(idx_hbm.at[pl.ds(off, _CS)], idx_v)
            pltpu.async_copy(table_hbm.at[idx_v], rows_v, sem).wait()
            pltpu.sync_copy(rows_v, out_hbm.at[pl.ds(off, _CS)])

    out = gather_k(data, flat_idx)
    return out.reshape(batch, hist, dim)


# speedup vs baseline: 1.1093x; 1.0097x over previous
import functools

import jax
import jax.numpy as jnp
from jax import lax
from jax.experimental import pallas as pl
from jax.experimental.pallas import tpu as pltpu
from jax.experimental.pallas import tpu_sc as plsc

_NC = 2
_NS = 16
_NW = _NC * _NS
_CS = 1600


def kernel(data, indices):
    batch, hist = indices.shape
    _, dim = data.shape
    n = batch * hist
    flat_idx = indices.reshape(n)
    b_per_w = n // _NW
    n_chunks = b_per_w // _CS

    mesh = plsc.VectorSubcoreMesh(core_axis_name="c", subcore_axis_name="s")

    @functools.partial(
        pl.kernel,
        mesh=mesh,
        compiler_params=pltpu.CompilerParams(use_tc_tiling_on_sc=False),
        out_type=jax.ShapeDtypeStruct((n, dim), jnp.float32),
        scratch_types=[
            pltpu.VMEM((b_per_w,), jnp.int32),
            pltpu.VMEM((_CS, dim), jnp.float32),
            pltpu.VMEM((_CS, dim), jnp.float32),
            pltpu.SemaphoreType.DMA,
            pltpu.SemaphoreType.DMA,
            pltpu.SemaphoreType.DMA,
            pltpu.SemaphoreType.DMA,
        ],
    )
    def gather_k(table_hbm, idx_hbm, out_hbm, idx_all, rows0, rows1,
                 sg0, sg1, ss0, ss1):
        wid = lax.axis_index("s") * _NC + lax.axis_index("c")
        base = wid * b_per_w
        rows = (rows0, rows1)
        sg = (sg0, sg1)
        ss = (ss0, ss1)

        def gather_start(j, b):
            pltpu.async_copy(
                table_hbm.at[idx_all.at[pl.ds(j * _CS, _CS)]], rows[b], sg[b])

        def gather_wait(b):
            pltpu.make_async_copy(
                table_hbm.at[idx_all.at[pl.ds(0, _CS)]], rows[b], sg[b]).wait()

        def store_start(j, b):
            pltpu.async_copy(rows[b], out_hbm.at[pl.ds(base + j * _CS, _CS)],
                             ss[b])

        def store_wait(b):
            pltpu.make_async_copy(rows[b], out_hbm.at[pl.ds(base, _CS)],
                                  ss[b]).wait()

        pltpu.sync_copy(idx_hbm.at[pl.ds(base, b_per_w)], idx_all)

        gather_start(0, 0)
        gather_wait(0)
        store_start(0, 0)
        gather_start(1, 1)

        @pl.loop(0, (n_chunks - 2) // 2)
        def _pipe(t):
            for k in range(2):
                j = 2 * t + 1 + k
                b = (1 + k) % 2
                gather_wait(b)
                store_start(j, b)
                store_wait(1 - b)
                gather_start(j + 1, 1 - b)

        bl = (n_chunks - 1) % 2
        gather_wait(bl)
        store_start(n_chunks - 1, bl)
        store_wait(1 - bl)
        store_wait(bl)

    out = gather_k(data, flat_idx)
    return out.reshape(batch, hist, dim)
